# Initial kernel scaffold; baseline (speedup 1.0000x reference)
import jax
import jax.numpy as jnp
from jax.experimental import pallas as pl


def kernel(tokens, token_table, position_table):
    raise NotImplementedError("write your pallas kernel here")


# speedup vs baseline: 1.4276x; 1.4276x over previous
import functools

import jax
import jax.numpy as jnp
from jax import lax
from jax.experimental import pallas as pl
from jax.experimental.pallas import tpu as pltpu
from jax.experimental.pallas import tpu_sc as plsc

VOCAB = 1_000_000
D = 32
SEQ = 200
BATCH = 4096
NTOK = BATCH * SEQ

NC, NS, L = 2, 16, 16
NW = NC * NS
B_PER_W = NTOK // NW
CHUNK = 1600
NCHUNK = B_PER_W // CHUNK
REP = CHUNK // SEQ


def _tpe_kernel(tok_table, idx_hbm, pos_hbm, out_hbm,
                idx_v, rows_v, pos_v, sem):
    wid = lax.axis_index("s") * NC + lax.axis_index("c")
    base = wid * B_PER_W

    pltpu.sync_copy(pos_hbm, pos_v)

    @pl.loop(0, NCHUNK)
    def _chunk(i):
        off = base + i * CHUNK
        pltpu.sync_copy(idx_hbm.at[pl.ds(off, CHUNK)], idx_v)
        pltpu.async_copy(tok_table.at[idx_v], rows_v, sem).wait()

        @pl.loop(0, SEQ)
        def _pos(l):
            p0 = pos_v[l, pl.ds(0, L)]
            p1 = pos_v[l, pl.ds(L, L)]
            for rep in range(REP):
                r = rep * SEQ + l
                plsc.addupdate(rows_v.at[r, pl.ds(0, L)], p0)
                plsc.addupdate(rows_v.at[r, pl.ds(L, L)], p1)

        pltpu.sync_copy(rows_v, out_hbm.at[pl.ds(off, CHUNK)])


@jax.jit
def _tpe(tokens_flat, token_table, position_table):
    mesh = plsc.VectorSubcoreMesh(core_axis_name="c", subcore_axis_name="s")
    run = pl.kernel(
        _tpe_kernel,
        out_type=jax.ShapeDtypeStruct((NTOK, D), jnp.float32),
        mesh=mesh,
        scratch_types=[
            pltpu.VMEM((CHUNK,), jnp.int32),
            pltpu.VMEM((CHUNK, D), jnp.float32),
            pltpu.VMEM((SEQ, D), jnp.float32),
            pltpu.SemaphoreType.DMA,
        ],
        compiler_params=pltpu.CompilerParams(use_tc_tiling_on_sc=False),
    )
    return run(token_table, tokens_flat, position_table)


def kernel(tokens, token_table, position_table):
    tokens_flat = tokens.reshape(-1).astype(jnp.int32)
    out = _tpe(tokens_flat, token_table, position_table)
    return out.reshape(BATCH, SEQ, D)
